# baseline (speedup 1.0000x reference)
import jax
import jax.numpy as jnp
from jax.experimental import pallas as pl
from jax.experimental.pallas import tpu as pltpu

_DIM = 64
_WIDE = 2 * _DIM
_LEVELS = 8
_EPS = 1e-8
_BLOCK = 8192


def _hilo_dot(a, m):
    a_hi = a.astype(jnp.bfloat16).astype(jnp.float32)
    a_lo = a - a_hi
    dn = (((1,), (0,)), ((), ()))
    return (jax.lax.dot_general(a_hi, m, dn, preferred_element_type=jnp.float32)
            + jax.lax.dot_general(a_lo, m, dn, preferred_element_type=jnp.float32))


def _pq_block(x_ref, a1_ref, w2_ref, cb_ref, cc_ref, o_ref):
    dn = (((1,), (0,)), ((), ()))

    y = jax.lax.dot_general(x_ref[...], a1_ref[...], dn,
                            preferred_element_type=jnp.float32)

    s = y * y
    jj = jax.lax.broadcasted_iota(jnp.int32, (_WIDE, _WIDE), 0)
    ii = jax.lax.broadcasted_iota(jnp.int32, (_WIDE, _WIDE), 1)
    same = (jj < _DIM) == (ii < _DIM)
    m_suffix = ((jj >= ii) & same).astype(jnp.float32)
    suffix2 = _hilo_dot(s, m_suffix)

    ct = y * jax.lax.rsqrt(suffix2 + 1e-30)

    cos_q = jnp.full_like(ct, cc_ref[0, 0])
    for k in range(1, _LEVELS):
        cos_q = jnp.where(ct < cb_ref[0, k - 1], cc_ref[0, k], cos_q)
    logsin_q = 0.5 * jnp.log1p(-cos_q * cos_q)

    col = jax.lax.broadcasted_iota(jnp.int32, cos_q.shape, 1)
    cos_q = jnp.where((col == _DIM - 1) | (col == _WIDE - 1), 1.0, cos_q)

    m_excl = ((jj < ii) & same).astype(jnp.float32)
    lcs = _hilo_dot(logsin_q, m_excl)
    xp = jnp.exp(lcs) * cos_q

    rt = jnp.sqrt(suffix2[:, 0:1]) + _EPS
    rb = jnp.sqrt(suffix2[:, _DIM:_DIM + 1]) + _EPS
    scale = jnp.where(col < _DIM, rt, rb)
    o_ref[...] = jax.lax.dot_general(xp * scale, w2_ref[...], dn,
                                     preferred_element_type=jnp.float32)


def kernel(x, D, H, centroids, boundaries):
    n = x.shape[0]
    zeros = jnp.zeros((_DIM, _DIM), jnp.float32)
    m1 = (H * D[None, :]).T.astype(jnp.float32)
    m2 = (H * D[None, :]).astype(jnp.float32)
    a1 = jnp.concatenate([
        jnp.concatenate([m1, zeros], axis=1),
        jnp.concatenate([zeros, m1], axis=1)], axis=0)
    w2 = jnp.concatenate([
        jnp.concatenate([m2, zeros], axis=1),
        jnp.concatenate([zeros, m2], axis=1)], axis=0)
    cos_b = jnp.cos(boundaries[1:_LEVELS]).reshape(1, _LEVELS - 1)
    cos_c = jnp.cos(centroids).reshape(1, _LEVELS)

    blk = min(_BLOCK, n)
    pad = (-n) % (2 * blk)
    xin = jnp.pad(x, ((0, pad), (0, 0))) if pad else x
    npad = n + pad
    x2 = xin.reshape(npad // 2, _WIDE)
    b2 = blk // 2

    out = pl.pallas_call(
        _pq_block,
        grid=(npad // (2 * b2),),
        in_specs=[
            pl.BlockSpec((b2, _WIDE), lambda g: (g, 0)),
            pl.BlockSpec((_WIDE, _WIDE), lambda g: (0, 0)),
            pl.BlockSpec((_WIDE, _WIDE), lambda g: (0, 0)),
            pl.BlockSpec(memory_space=pltpu.SMEM),
            pl.BlockSpec(memory_space=pltpu.SMEM),
        ],
        out_specs=pl.BlockSpec((b2, _WIDE), lambda g: (g, 0)),
        out_shape=jax.ShapeDtypeStruct((npad // 2, _WIDE), jnp.float32),
        compiler_params=pltpu.CompilerParams(
            dimension_semantics=("parallel",)),
    )(x2, a1, w2, cos_b, cos_c)
    out = out.reshape(npad, _DIM)
    return out[:n] if pad else out


# speedup vs baseline: 1.3576x; 1.3576x over previous
import jax
import jax.numpy as jnp
from jax.experimental import pallas as pl
from jax.experimental.pallas import tpu as pltpu

_DIM = 64
_WIDE = 2 * _DIM
_LEVELS = 8
_EPS = 1e-8
_BLOCK = 8192


def _hilo_dot(a, m):
    a_hi = a.astype(jnp.bfloat16).astype(jnp.float32)
    a_lo = a - a_hi
    dn = (((1,), (0,)), ((), ()))
    return (jax.lax.dot_general(a_hi, m, dn, preferred_element_type=jnp.float32)
            + jax.lax.dot_general(a_lo, m, dn, preferred_element_type=jnp.float32))


def _pq_block(x_ref, a1l_ref, a1r_ref, w2t_ref, w2b_ref,
              cb_ref, cc_ref, ls_ref, o_ref):
    b2 = x_ref.shape[0] // 2
    xt = x_ref[:b2]
    xb = x_ref[b2:]
    dn = (((1,), (0,)), ((), ()))

    y = (jax.lax.dot_general(xt, a1l_ref[...], dn,
                             preferred_element_type=jnp.float32)
         + jax.lax.dot_general(xb, a1r_ref[...], dn,
                               preferred_element_type=jnp.float32))

    s = y * y
    jj = jax.lax.broadcasted_iota(jnp.int32, (_WIDE, _WIDE), 0)
    ii = jax.lax.broadcasted_iota(jnp.int32, (_WIDE, _WIDE), 1)
    same = (jj < _DIM) == (ii < _DIM)
    m_suffix = ((jj >= ii) & same).astype(jnp.float32)
    suffix2 = _hilo_dot(s, m_suffix)
    rem = jnp.sqrt(suffix2) + _EPS

    ct = y / rem

    cos_q = jnp.full_like(ct, cc_ref[0, 0])
    logsin_q = jnp.full_like(ct, ls_ref[0, 0])
    for k in range(1, _LEVELS):
        mask = ct < cb_ref[0, k - 1]
        cos_q = jnp.where(mask, cc_ref[0, k], cos_q)
        logsin_q = jnp.where(mask, ls_ref[0, k], logsin_q)

    col = jax.lax.broadcasted_iota(jnp.int32, cos_q.shape, 1)
    cos_q = jnp.where((col == _DIM - 1) | (col == _WIDE - 1), 1.0, cos_q)

    m_excl = ((jj < ii) & same).astype(jnp.float32)
    lcs = _hilo_dot(logsin_q, m_excl)
    xp = jnp.exp(lcs) * cos_q

    o_ref[:b2] = jax.lax.dot_general(
        xp, w2t_ref[...], dn,
        preferred_element_type=jnp.float32) * rem[:, 0:1]
    o_ref[b2:] = jax.lax.dot_general(
        xp, w2b_ref[...], dn,
        preferred_element_type=jnp.float32) * rem[:, _DIM:_DIM + 1]


def kernel(x, D, H, centroids, boundaries):
    n = x.shape[0]
    zeros = jnp.zeros((_DIM, _DIM), jnp.float32)
    m1 = (H * D[None, :]).T.astype(jnp.float32)
    m2 = (H * D[None, :]).astype(jnp.float32)
    a1l = jnp.concatenate([m1, zeros], axis=1)
    a1r = jnp.concatenate([zeros, m1], axis=1)
    w2t = jnp.concatenate([m2, zeros], axis=0)
    w2b = jnp.concatenate([zeros, m2], axis=0)
    cos_b = jnp.cos(boundaries[1:_LEVELS]).reshape(1, _LEVELS - 1)
    cos_c = jnp.cos(centroids).reshape(1, _LEVELS)
    logsin_c = jnp.log(jnp.sin(centroids)).reshape(1, _LEVELS)

    blk = min(_BLOCK, n)
    pad = (-n) % blk
    xin = jnp.pad(x, ((0, pad), (0, 0))) if pad else x
    npad = n + pad

    out = pl.pallas_call(
        _pq_block,
        grid=(npad // blk,),
        in_specs=[
            pl.BlockSpec((blk, _DIM), lambda g: (g, 0)),
            pl.BlockSpec((_DIM, _WIDE), lambda g: (0, 0)),
            pl.BlockSpec((_DIM, _WIDE), lambda g: (0, 0)),
            pl.BlockSpec((_WIDE, _DIM), lambda g: (0, 0)),
            pl.BlockSpec((_WIDE, _DIM), lambda g: (0, 0)),
            pl.BlockSpec(memory_space=pltpu.SMEM),
            pl.BlockSpec(memory_space=pltpu.SMEM),
            pl.BlockSpec(memory_space=pltpu.SMEM),
        ],
        out_specs=pl.BlockSpec((blk, _DIM), lambda g: (g, 0)),
        out_shape=jax.ShapeDtypeStruct((npad, _DIM), jnp.float32),
        compiler_params=pltpu.CompilerParams(
            dimension_semantics=("parallel",)),
    )(xin, a1l, a1r, w2t, w2b, cos_b, cos_c, logsin_c)
    return out[:n] if pad else out
